# baseline (speedup 1.0000x reference)
import jax
import jax.numpy as jnp
from jax import lax
from jax.experimental import pallas as pl
from jax.experimental.pallas import tpu as pltpu
from jax.experimental.pallas import tpu_sc as plsc

N_NODES = 10000
D_FEAT = 128
N_EDGES = 320000
NC = 2
NS = 16
NW = NC * NS
EPW = N_EDGES // NW
K = 128
NCHUNK = 80
HALF = NCHUNK // 2
PAIRS_PER_HALF = HALF // 2
EPW_PAD = NCHUNK * K
ACC_ROWS = N_NODES + 8
ROWS_PER_TILE = (N_NODES // NS) // 8 * 8
ROWS_TAIL = N_NODES - NS * ROWS_PER_TILE


def _sc_partial_body(x_hbm, src_hbm, dst_hbm, p_hbm,
                     src_v, dst_v, rows0, rows1,
                     acc, sem0, sem1):
    cid = lax.axis_index("c")
    sid = lax.axis_index("s")
    wid = cid * NS + sid

    r0 = sid * ROWS_PER_TILE
    pltpu.sync_copy(x_hbm.at[pl.ds(r0, ROWS_PER_TILE)],
                    acc.at[pl.ds(r0, ROWS_PER_TILE)])

    @pl.when(sid == NS - 1)
    def _init_tail():
        t0 = NS * ROWS_PER_TILE
        pltpu.sync_copy(x_hbm.at[pl.ds(t0, ROWS_TAIL)],
                        acc.at[pl.ds(t0, ROWS_TAIL)])

    plsc.subcore_barrier()

    for h in (0, 1):
        pltpu.sync_copy(src_hbm.at[wid, pl.ds(h * HALF, HALF)], src_v)
        pltpu.sync_copy(dst_hbm.at[wid, pl.ds(h * HALF, HALF)], dst_v)
        pltpu.async_copy(x_hbm.at[src_v.at[0]], rows0, sem0)

        def pair(i, carry):
            j0 = 2 * i
            j1 = j0 + 1
            pltpu.make_async_copy(x_hbm.at[src_v.at[j0]], rows0, sem0).wait()
            pltpu.async_copy(x_hbm.at[src_v.at[j1]], rows1, sem1)
            pltpu.sync_copy(rows0, acc.at[dst_v.at[j0]], add=True)
            pltpu.make_async_copy(x_hbm.at[src_v.at[j1]], rows1, sem1).wait()

            @pl.when(i < PAIRS_PER_HALF - 1)
            def _next():
                pltpu.async_copy(x_hbm.at[src_v.at[j0 + 2]], rows0, sem0)

            pltpu.sync_copy(rows1, acc.at[dst_v.at[j1]], add=True)
            return carry

        lax.fori_loop(0, PAIRS_PER_HALF, pair, 0)

    plsc.subcore_barrier()
    pltpu.sync_copy(acc.at[pl.ds(r0, ROWS_PER_TILE)],
                    p_hbm.at[cid, pl.ds(r0, ROWS_PER_TILE)])

    @pl.when(sid == NS - 1)
    def _drain_tail():
        t0 = NS * ROWS_PER_TILE
        pltpu.sync_copy(acc.at[pl.ds(t0, ROWS_TAIL)],
                        p_hbm.at[cid, pl.ds(t0, ROWS_TAIL)])


def _combine_body(x_ref, p_ref, o_ref):
    o_ref[...] = p_ref[0] + p_ref[1] - x_ref[...]


def kernel(x, edge_index):
    src = edge_index[0].astype(jnp.int32).reshape(NW, EPW)
    dst = edge_index[1].astype(jnp.int32).reshape(NW, EPW)
    pad = EPW_PAD - EPW
    src3 = jnp.pad(src, ((0, 0), (0, pad))).reshape(NW, NCHUNK, K)
    dst3 = jnp.pad(dst, ((0, 0), (0, pad)),
                   constant_values=N_NODES).reshape(NW, NCHUNK, K)

    mesh = plsc.VectorSubcoreMesh(core_axis_name="c", subcore_axis_name="s",
                                  num_cores=NC, num_subcores=NS)
    p = pl.kernel(
        _sc_partial_body,
        out_type=jax.ShapeDtypeStruct((NC, N_NODES, D_FEAT), jnp.float32),
        mesh=mesh,
        scratch_types=[
            pltpu.VMEM((HALF, K), jnp.int32),
            pltpu.VMEM((HALF, K), jnp.int32),
            pltpu.VMEM((K, D_FEAT), jnp.float32),
            pltpu.VMEM((K, D_FEAT), jnp.float32),
            pltpu.VMEM_SHARED((ACC_ROWS, D_FEAT), jnp.float32),
            pltpu.SemaphoreType.DMA,
            pltpu.SemaphoreType.DMA,
        ],
    )(x, src3, dst3)

    BLK = 400
    out = pl.pallas_call(
        _combine_body,
        out_shape=jax.ShapeDtypeStruct((N_NODES, D_FEAT), jnp.float32),
        grid=(N_NODES // BLK,),
        in_specs=[pl.BlockSpec((BLK, D_FEAT), lambda i: (i, 0)),
                  pl.BlockSpec((NC, BLK, D_FEAT), lambda i: (0, i, 0))],
        out_specs=pl.BlockSpec((BLK, D_FEAT), lambda i: (i, 0)),
    )(x, p)
    return out
